# baseline (speedup 1.0000x reference)
import functools

import jax
import jax.numpy as jnp
from jax import lax
from jax.experimental import pallas as pl
from jax.experimental.pallas import tpu as pltpu
from jax.experimental.pallas import tpu_sc as plsc

N = 10000
E = 320000
D = 128

NC = 2
NS = 16
NW = NC * NS
EPW = E // NW
KJ = 80
NJ = EPW // KJ
SLAB = 624
ZR = 16
REM = N - NS * SLAB


def _build_sc_agg(nrel, with_deg):
    mesh = plsc.VectorSubcoreMesh(core_axis_name="c", subcore_axis_name="s")
    out_type = [jax.ShapeDtypeStruct((nrel, NC, N, D), jnp.float32)]
    if with_deg:
        out_type.append(jax.ShapeDtypeStruct((nrel * NC * N,), jnp.float32))
    scratch = [
        pltpu.VMEM((EPW,), jnp.int32),
        pltpu.VMEM((NJ, KJ), jnp.int32),
        pltpu.VMEM((KJ, D), jnp.float32),
        pltpu.VMEM((KJ, D), jnp.float32),
        pltpu.VMEM((ZR, D), jnp.float32),
        pltpu.VMEM_SHARED((N, D), jnp.float32),
        pltpu.SemaphoreType.DMA,
        pltpu.SemaphoreType.DMA,
        pltpu.SemaphoreType.DMA,
        pltpu.SemaphoreType.DMA,
    ]
    if with_deg:
        scratch += [
            pltpu.VMEM((KJ,), jnp.float32),
            pltpu.VMEM((1024,), jnp.float32),
            pltpu.VMEM_SHARED((N,), jnp.float32),
            pltpu.SemaphoreType.DMA,
        ]

    @functools.partial(pl.kernel, out_type=tuple(out_type), mesh=mesh,
                       scratch_types=scratch)
    def sc_agg(*refs):
        if with_deg:
            (tab_hbm, srcf_hbm, dstb_hbm, agg_out, deg_out,
             src_v, dst_v, rows_a, rows_b, zer_v, agg_sh,
             gsem_a, gsem_b, ssem_a, ssem_b,
             ones_v, dz_v, deg_sh, dsem) = refs
        else:
            (tab_hbm, srcf_hbm, dstb_hbm, agg_out,
             src_v, dst_v, rows_a, rows_b, zer_v, agg_sh,
             gsem_a, gsem_b, ssem_a, ssem_b) = refs

        c = lax.axis_index("c")
        s = lax.axis_index("s")
        w = c * NS + s

        zero16 = jnp.zeros((16,), jnp.float32)
        cols = D // 16

        def fill_zer(i, carry):
            zer_v[i // cols, pl.ds((i % cols) * 16, 16)] = zero16
            return carry
        lax.fori_loop(0, ZR * cols, fill_zer, 0)

        if with_deg:
            def fill_dz(i, carry):
                dz_v[pl.ds(i * 16, 16)] = zero16
                return carry

            lax.fori_loop(0, 1024 // 16, fill_dz, 0)

            one16 = jnp.ones((16,), jnp.float32)

            def fill_ones(i, carry):
                ones_v[pl.ds(i * 16, 16)] = one16
                return carry
            lax.fori_loop(0, KJ // 16, fill_ones, 0)

        for r in range(nrel):
            for b in range(SLAB // ZR):
                off = pl.multiple_of(s * SLAB + b * ZR, 8)
                pltpu.sync_copy(zer_v, agg_sh.at[pl.ds(off, ZR), :])

            @pl.when(s == 0)
            def _():
                pltpu.sync_copy(zer_v.at[pl.ds(0, REM), :],
                                agg_sh.at[pl.ds(NS * SLAB, REM), :])
            if with_deg:
                @pl.when(s < 10)
                def _():
                    off = pl.multiple_of(s * 1000, 8)
                    pltpu.sync_copy(dz_v.at[pl.ds(0, 1000)],
                                    deg_sh.at[pl.ds(off, 1000)])
            plsc.subcore_barrier()

            soff0 = pl.multiple_of(r * E + w * EPW, 8)
            pltpu.sync_copy(srcf_hbm.at[pl.ds(soff0, EPW)], src_v)
            pltpu.sync_copy(dstb_hbm.at[r, w], dst_v)

            def gstart(j, buf, sem):
                idx = src_v.at[pl.ds(pl.multiple_of(j * KJ, 8), KJ)]
                pltpu.make_async_copy(tab_hbm.at[idx], buf, sem).start()

            def gwait(j, buf, sem):
                idx = src_v.at[pl.ds(pl.multiple_of(j * KJ, 8), KJ)]
                pltpu.make_async_copy(tab_hbm.at[idx], buf, sem).wait()

            def sstart(j, buf, sem):
                pltpu.async_copy(buf, agg_sh.at[dst_v.at[j]], sem, add=True)
                if with_deg:
                    pltpu.async_copy(ones_v, deg_sh.at[dst_v.at[j]], dsem,
                                     add=True)

            def swait(j, buf, sem):
                pltpu.make_async_copy(buf, agg_sh.at[dst_v.at[j]], sem).wait()
                if with_deg:
                    pltpu.make_async_copy(ones_v, deg_sh.at[dst_v.at[j]],
                                          dsem).wait()

            gstart(0, rows_a, gsem_a)
            gstart(1, rows_b, gsem_b)

            def pair(p, carry):
                j0 = p * 2
                gwait(j0, rows_a, gsem_a)
                sstart(j0, rows_a, ssem_a)
                gwait(j0 + 1, rows_b, gsem_b)
                sstart(j0 + 1, rows_b, ssem_b)
                swait(j0, rows_a, ssem_a)
                gstart(j0 + 2, rows_a, gsem_a)
                swait(j0 + 1, rows_b, ssem_b)

                @pl.when(j0 + 3 < NJ)
                def _():
                    gstart(j0 + 3, rows_b, gsem_b)
                return carry
            lax.fori_loop(0, (NJ - 1) // 2, pair, 0)

            gwait(NJ - 1, rows_a, gsem_a)
            pltpu.sync_copy(rows_a, agg_sh.at[dst_v.at[NJ - 1]], add=True)
            if with_deg:
                pltpu.sync_copy(ones_v, deg_sh.at[dst_v.at[NJ - 1]], add=True)

            plsc.subcore_barrier()

            soff = pl.multiple_of(s * SLAB, 8)
            pltpu.sync_copy(agg_sh.at[pl.ds(soff, SLAB), :],
                            agg_out.at[r, c, pl.ds(soff, SLAB), :])

            @pl.when(s == 0)
            def _():
                pltpu.sync_copy(agg_sh.at[pl.ds(NS * SLAB, REM), :],
                                agg_out.at[r, c, pl.ds(NS * SLAB, REM), :])
            if with_deg:
                @pl.when(s < 10)
                def _():
                    off = pl.multiple_of(s * 1000, 8)
                    doff = pl.multiple_of((r * NC + c) * N + s * 1000, 8)
                    pltpu.sync_copy(deg_sh.at[pl.ds(off, 1000)],
                                    dz_v.at[pl.ds(0, 1000)])
                    pltpu.sync_copy(dz_v.at[pl.ds(0, 1000)],
                                    deg_out.at[pl.ds(doff, 1000)])
                    if r + 1 < nrel:
                        def refill(i, carry):
                            dz_v[pl.ds(i * 16, 16)] = jnp.zeros(
                                (16,), jnp.float32)
                            return carry
                        lax.fori_loop(0, 1024 // 16, refill, 0)
            if r + 1 < nrel:
                plsc.subcore_barrier()

    return sc_agg


_sc_agg3 = _build_sc_agg(3, True)
_sc_agg1 = _build_sc_agg(1, False)

RB = 1000


def _tc_layer1(ap, dp, x, wl, bl, wr):
    def body(ap_ref, dp_ref, x_ref, wl_ref, bl_ref, wr_ref, o_ref):
        xb = x_ref[...]
        acc = jnp.zeros((RB, D), jnp.float32)
        for r in range(3):
            agg = ap_ref[r, 0] + ap_ref[r, 1]
            deg = jnp.maximum(dp_ref[r, 0] + dp_ref[r, 1], 1.0)
            agg = agg / deg
            v = (jnp.dot(agg, wl_ref[r], preferred_element_type=jnp.float32)
                 + jnp.dot(xb, wr_ref[r], preferred_element_type=jnp.float32)
                 + bl_ref[r][None, :])
            acc = acc + jnp.maximum(v, 0.0)
        o_ref[...] = acc * (1.0 / 3.0)

    return pl.pallas_call(
        body,
        grid=(N // RB,),
        in_specs=[
            pl.BlockSpec((3, NC, RB, D), lambda i: (0, 0, i, 0)),
            pl.BlockSpec((3, NC, RB, 1), lambda i: (0, 0, i, 0)),
            pl.BlockSpec((RB, D), lambda i: (i, 0)),
            pl.BlockSpec((3, D, D), lambda i: (0, 0, 0)),
            pl.BlockSpec((3, D), lambda i: (0, 0)),
            pl.BlockSpec((3, D, D), lambda i: (0, 0, 0)),
        ],
        out_specs=pl.BlockSpec((RB, D), lambda i: (i, 0)),
        out_shape=jax.ShapeDtypeStruct((N, D), jnp.float32),
    )(ap, dp, x, wl, bl, wr)


def _tc_layer2(af, dg, h, wlf, blf, wrf):
    def body(af_ref, dg_ref, h_ref, wl_ref, bl_ref, wr_ref, o_ref):
        agg = af_ref[0] + af_ref[1]
        deg = jnp.maximum(dg_ref[0] + dg_ref[1], 1.0)
        agg = agg / deg
        o_ref[...] = (jnp.dot(agg, wl_ref[...], preferred_element_type=jnp.float32)
                      + jnp.dot(h_ref[...], wr_ref[...], preferred_element_type=jnp.float32)
                      + bl_ref[...])

    return pl.pallas_call(
        body,
        grid=(N // RB,),
        in_specs=[
            pl.BlockSpec((NC, RB, D), lambda i: (0, i, 0)),
            pl.BlockSpec((NC, RB, 1), lambda i: (0, i, 0)),
            pl.BlockSpec((RB, D), lambda i: (i, 0)),
            pl.BlockSpec((D, D), lambda i: (0, 0)),
            pl.BlockSpec((1, D), lambda i: (0, 0)),
            pl.BlockSpec((D, D), lambda i: (0, 0)),
        ],
        out_specs=pl.BlockSpec((RB, D), lambda i: (i, 0)),
        out_shape=jax.ShapeDtypeStruct((N, D), jnp.float32),
    )(af, dg, h, wlf, blf, wrf)


def kernel(x, edge_index_0, edge_index_1, edge_index_2,
           W_l_0, b_l_0, W_r_0,
           W_l_1, b_l_1, W_r_1,
           W_l_2, b_l_2, W_r_2,
           W_l_f, b_l_f, W_r_f):
    e_all = jnp.stack([edge_index_0, edge_index_1, edge_index_2])
    e_all = e_all.astype(jnp.int32)
    src_flat = e_all[:, 0, :].reshape(3 * E)
    dst_blk = e_all[:, 1, :].reshape(3, NW, NJ, KJ)
    agg3, deg3 = _sc_agg3(x, src_flat, dst_blk)
    deg3 = deg3.reshape(3, NC, N, 1)

    wl = jnp.stack([W_l_0, W_l_1, W_l_2])
    bl = jnp.stack([b_l_0, b_l_1, b_l_2])
    wr = jnp.stack([W_r_0, W_r_1, W_r_2])
    h = _tc_layer1(agg3, deg3, x, wl, bl, wr)

    ef = edge_index_0.astype(jnp.int32)
    (aggf,) = _sc_agg1(h, ef[0], ef[1].reshape(1, NW, NJ, KJ))

    out = _tc_layer2(aggf[0], deg3[0], h,
                     W_l_f, b_l_f.reshape(1, D), W_r_f)
    return out


# speedup vs baseline: 1.1491x; 1.1491x over previous
import functools

import jax
import jax.numpy as jnp
from jax import lax
from jax.experimental import pallas as pl
from jax.experimental.pallas import tpu as pltpu
from jax.experimental.pallas import tpu_sc as plsc

N = 10000
E = 320000
D = 128

NC = 2
NS = 16
NW = NC * NS
EPW = E // NW
KJ = 80
NJ = EPW // KJ
SLAB = 624
ZR = 16
REM = N - NS * SLAB


def _build_sc_agg(nrel, with_deg):
    mesh = plsc.VectorSubcoreMesh(core_axis_name="c", subcore_axis_name="s")
    out_type = [jax.ShapeDtypeStruct((nrel, NC, N, D), jnp.float32)]
    if with_deg:
        out_type.append(jax.ShapeDtypeStruct((nrel * NC * N,), jnp.float32))
    scratch = [
        pltpu.VMEM((EPW,), jnp.int32),
        pltpu.VMEM((NJ, KJ), jnp.int32),
        pltpu.VMEM((KJ, D), jnp.float32),
        pltpu.VMEM((KJ, D), jnp.float32),
        pltpu.VMEM((ZR, D), jnp.float32),
        pltpu.VMEM_SHARED((N, D), jnp.float32),
        pltpu.SemaphoreType.DMA,
        pltpu.SemaphoreType.DMA,
    ]
    if with_deg:
        scratch += [
            pltpu.VMEM((KJ,), jnp.float32),
            pltpu.VMEM((1024,), jnp.float32),
            pltpu.VMEM_SHARED((N,), jnp.float32),
            pltpu.SemaphoreType.DMA,
        ]

    @functools.partial(pl.kernel, out_type=tuple(out_type), mesh=mesh,
                       scratch_types=scratch)
    def sc_agg(*refs):
        if with_deg:
            (tab_hbm, srcf_hbm, dstb_hbm, agg_out, deg_out,
             src_v, dst_v, rows_a, rows_b, zer_v, agg_sh,
             gsem_a, gsem_b,
             ones_v, dz_v, deg_sh, dsem) = refs
        else:
            (tab_hbm, srcf_hbm, dstb_hbm, agg_out,
             src_v, dst_v, rows_a, rows_b, zer_v, agg_sh,
             gsem_a, gsem_b) = refs

        c = lax.axis_index("c")
        s = lax.axis_index("s")
        w = c * NS + s

        zero16 = jnp.zeros((16,), jnp.float32)
        cols = D // 16

        def fill_zer(i, carry):
            zer_v[i // cols, pl.ds((i % cols) * 16, 16)] = zero16
            return carry
        lax.fori_loop(0, ZR * cols, fill_zer, 0)

        if with_deg:
            def fill_dz(i, carry):
                dz_v[pl.ds(i * 16, 16)] = zero16
                return carry

            lax.fori_loop(0, 1024 // 16, fill_dz, 0)

            one16 = jnp.ones((16,), jnp.float32)

            def fill_ones(i, carry):
                ones_v[pl.ds(i * 16, 16)] = one16
                return carry
            lax.fori_loop(0, KJ // 16, fill_ones, 0)

        for r in range(nrel):
            for b in range(SLAB // ZR):
                off = pl.multiple_of(s * SLAB + b * ZR, 8)
                pltpu.sync_copy(zer_v, agg_sh.at[pl.ds(off, ZR), :])

            @pl.when(s == 0)
            def _():
                pltpu.sync_copy(zer_v.at[pl.ds(0, REM), :],
                                agg_sh.at[pl.ds(NS * SLAB, REM), :])
            if with_deg:
                @pl.when(s < 10)
                def _():
                    off = pl.multiple_of(s * 1000, 8)
                    pltpu.sync_copy(dz_v.at[pl.ds(0, 1000)],
                                    deg_sh.at[pl.ds(off, 1000)])
            plsc.subcore_barrier()

            soff0 = pl.multiple_of(r * E + w * EPW, 8)
            pltpu.sync_copy(srcf_hbm.at[pl.ds(soff0, EPW)], src_v)
            pltpu.sync_copy(dstb_hbm.at[r, w], dst_v)

            def gstart(j, buf, sem):
                idx = src_v.at[pl.ds(pl.multiple_of(j * KJ, 8), KJ)]
                pltpu.make_async_copy(tab_hbm.at[idx], buf, sem).start()

            def gwait(j, buf, sem):
                idx = src_v.at[pl.ds(pl.multiple_of(j * KJ, 8), KJ)]
                pltpu.make_async_copy(tab_hbm.at[idx], buf, sem).wait()

            def drain(j, buf, sem):
                gwait(j, buf, sem)
                pltpu.sync_copy(buf, agg_sh.at[dst_v.at[j]], add=True)
                if with_deg:
                    pltpu.async_copy(ones_v, deg_sh.at[dst_v.at[j]], dsem,
                                     add=True)

            gstart(0, rows_a, gsem_a)

            def chunk2(p, carry):
                j0 = p * 2
                gstart(j0 + 1, rows_b, gsem_b)
                drain(j0, rows_a, gsem_a)
                gstart(j0 + 2, rows_a, gsem_a)
                drain(j0 + 1, rows_b, gsem_b)
                return carry
            lax.fori_loop(0, (NJ - 1) // 2, chunk2, 0)
            drain(NJ - 1, rows_a, gsem_a)

            if with_deg:
                def ddrain(j, carry):
                    pltpu.make_async_copy(ones_v, deg_sh.at[dst_v.at[j]],
                                          dsem).wait()
                    return carry
                lax.fori_loop(0, NJ, ddrain, 0)

            plsc.subcore_barrier()

            soff = pl.multiple_of(s * SLAB, 8)
            pltpu.sync_copy(agg_sh.at[pl.ds(soff, SLAB), :],
                            agg_out.at[r, c, pl.ds(soff, SLAB), :])

            @pl.when(s == 0)
            def _():
                pltpu.sync_copy(agg_sh.at[pl.ds(NS * SLAB, REM), :],
                                agg_out.at[r, c, pl.ds(NS * SLAB, REM), :])
            if with_deg:
                @pl.when(s < 10)
                def _():
                    off = pl.multiple_of(s * 1000, 8)
                    doff = pl.multiple_of((r * NC + c) * N + s * 1000, 8)
                    pltpu.sync_copy(deg_sh.at[pl.ds(off, 1000)],
                                    dz_v.at[pl.ds(0, 1000)])
                    pltpu.sync_copy(dz_v.at[pl.ds(0, 1000)],
                                    deg_out.at[pl.ds(doff, 1000)])
                    if r + 1 < nrel:
                        def refill(i, carry):
                            dz_v[pl.ds(i * 16, 16)] = jnp.zeros(
                                (16,), jnp.float32)
                            return carry
                        lax.fori_loop(0, 1024 // 16, refill, 0)
            if r + 1 < nrel:
                plsc.subcore_barrier()

    return sc_agg


_sc_agg3 = _build_sc_agg(3, True)
_sc_agg1 = _build_sc_agg(1, False)

RB = 1000


def _tc_layer1(ap, dp, x, wl, bl, wr):
    def body(ap_ref, dp_ref, x_ref, wl_ref, bl_ref, wr_ref, o_ref):
        xb = x_ref[...]
        acc = jnp.zeros((RB, D), jnp.float32)
        for r in range(3):
            agg = ap_ref[r, 0] + ap_ref[r, 1]
            deg = jnp.maximum(dp_ref[r, 0] + dp_ref[r, 1], 1.0)
            agg = agg / deg
            v = (jnp.dot(agg, wl_ref[r], preferred_element_type=jnp.float32)
                 + jnp.dot(xb, wr_ref[r], preferred_element_type=jnp.float32)
                 + bl_ref[r][None, :])
            acc = acc + jnp.maximum(v, 0.0)
        o_ref[...] = acc * (1.0 / 3.0)

    return pl.pallas_call(
        body,
        grid=(N // RB,),
        in_specs=[
            pl.BlockSpec((3, NC, RB, D), lambda i: (0, 0, i, 0)),
            pl.BlockSpec((3, NC, RB, 1), lambda i: (0, 0, i, 0)),
            pl.BlockSpec((RB, D), lambda i: (i, 0)),
            pl.BlockSpec((3, D, D), lambda i: (0, 0, 0)),
            pl.BlockSpec((3, D), lambda i: (0, 0)),
            pl.BlockSpec((3, D, D), lambda i: (0, 0, 0)),
        ],
        out_specs=pl.BlockSpec((RB, D), lambda i: (i, 0)),
        out_shape=jax.ShapeDtypeStruct((N, D), jnp.float32),
    )(ap, dp, x, wl, bl, wr)


def _tc_layer2(af, dg, h, wlf, blf, wrf):
    def body(af_ref, dg_ref, h_ref, wl_ref, bl_ref, wr_ref, o_ref):
        agg = af_ref[0] + af_ref[1]
        deg = jnp.maximum(dg_ref[0] + dg_ref[1], 1.0)
        agg = agg / deg
        o_ref[...] = (jnp.dot(agg, wl_ref[...], preferred_element_type=jnp.float32)
                      + jnp.dot(h_ref[...], wr_ref[...], preferred_element_type=jnp.float32)
                      + bl_ref[...])

    return pl.pallas_call(
        body,
        grid=(N // RB,),
        in_specs=[
            pl.BlockSpec((NC, RB, D), lambda i: (0, i, 0)),
            pl.BlockSpec((NC, RB, 1), lambda i: (0, i, 0)),
            pl.BlockSpec((RB, D), lambda i: (i, 0)),
            pl.BlockSpec((D, D), lambda i: (0, 0)),
            pl.BlockSpec((1, D), lambda i: (0, 0)),
            pl.BlockSpec((D, D), lambda i: (0, 0)),
        ],
        out_specs=pl.BlockSpec((RB, D), lambda i: (i, 0)),
        out_shape=jax.ShapeDtypeStruct((N, D), jnp.float32),
    )(af, dg, h, wlf, blf, wrf)


def kernel(x, edge_index_0, edge_index_1, edge_index_2,
           W_l_0, b_l_0, W_r_0,
           W_l_1, b_l_1, W_r_1,
           W_l_2, b_l_2, W_r_2,
           W_l_f, b_l_f, W_r_f):
    e_all = jnp.stack([edge_index_0, edge_index_1, edge_index_2])
    e_all = e_all.astype(jnp.int32)
    src_flat = e_all[:, 0, :].reshape(3 * E)
    dst_blk = e_all[:, 1, :].reshape(3, NW, NJ, KJ)
    agg3, deg3 = _sc_agg3(x, src_flat, dst_blk)
    deg3 = deg3.reshape(3, NC, N, 1)

    wl = jnp.stack([W_l_0, W_l_1, W_l_2])
    bl = jnp.stack([b_l_0, b_l_1, b_l_2])
    wr = jnp.stack([W_r_0, W_r_1, W_r_2])
    h = _tc_layer1(agg3, deg3, x, wl, bl, wr)

    ef = edge_index_0.astype(jnp.int32)
    (aggf,) = _sc_agg1(h, ef[0], ef[1].reshape(1, NW, NJ, KJ))

    out = _tc_layer2(aggf[0], deg3[0], h,
                     W_l_f, b_l_f.reshape(1, D), W_r_f)
    return out


# speedup vs baseline: 1.2528x; 1.0902x over previous
import functools

import jax
import jax.numpy as jnp
from jax import lax
from jax.experimental import pallas as pl
from jax.experimental.pallas import tpu as pltpu
from jax.experimental.pallas import tpu_sc as plsc

N = 10000
E = 320000
D = 128

NC = 2
NS = 16
NW = NC * NS
EPW = E // NW
KJ = 80
NJ = EPW // KJ
SLAB = 624
REM = N - NS * SLAB
NQ = 5
CQ = NJ // NQ
EQ = CQ * KJ


def _build_sc_agg(nrel, with_deg):
    mesh = plsc.VectorSubcoreMesh(core_axis_name="c", subcore_axis_name="s")
    out_type = [jax.ShapeDtypeStruct((nrel, NC, N, D), jnp.float32)]
    if with_deg:
        out_type.append(jax.ShapeDtypeStruct((nrel * NC * N,), jnp.float32))
    scratch = [
        pltpu.VMEM((EQ,), jnp.int32),
        pltpu.VMEM((EQ,), jnp.int32),
        pltpu.VMEM((CQ, KJ), jnp.int32),
        pltpu.VMEM((CQ, KJ), jnp.int32),
        pltpu.VMEM((KJ, D), jnp.float32),
        pltpu.VMEM((KJ, D), jnp.float32),
        pltpu.VMEM((KJ, D), jnp.float32),
        pltpu.VMEM_SHARED((N, D), jnp.float32),
        pltpu.SemaphoreType.DMA,
        pltpu.SemaphoreType.DMA,
        pltpu.SemaphoreType.DMA,
        pltpu.SemaphoreType.DMA,
        pltpu.SemaphoreType.DMA,
        pltpu.SemaphoreType.DMA,
        pltpu.SemaphoreType.DMA,
        pltpu.SemaphoreType.DMA,
    ]
    if with_deg:
        scratch += [
            pltpu.VMEM((KJ,), jnp.float32),
            pltpu.VMEM((1024,), jnp.float32),
            pltpu.VMEM_SHARED((N,), jnp.float32),
            pltpu.SemaphoreType.DMA,
        ]

    @functools.partial(pl.kernel, out_type=tuple(out_type), mesh=mesh,
                       scratch_types=scratch)
    def sc_agg(*refs):
        if with_deg:
            (tab_hbm, z2_hbm, z1_hbm, srcf_hbm, dstb_hbm, agg_out, deg_out,
             src_q0, src_q1, dst_q0, dst_q1, rows_0, rows_1, rows_2, agg_sh,
             gsem_0, gsem_1, gsem_2, ssem_0, ssem_1, ssem_2, isem_0, isem_1,
             ones_v, dstage_v, deg_sh, dsem) = refs
        else:
            (tab_hbm, z2_hbm, z1_hbm, srcf_hbm, dstb_hbm, agg_out,
             src_q0, src_q1, dst_q0, dst_q1, rows_0, rows_1, rows_2, agg_sh,
             gsem_0, gsem_1, gsem_2, ssem_0, ssem_1, ssem_2,
             isem_0, isem_1) = refs

        bufs = (rows_0, rows_1, rows_2)
        gsems = (gsem_0, gsem_1, gsem_2)
        ssems = (ssem_0, ssem_1, ssem_2)
        src_q = (src_q0, src_q1)
        dst_q = (dst_q0, dst_q1)
        isems = (isem_0, isem_1)

        c = lax.axis_index("c")
        s = lax.axis_index("s")
        w = c * NS + s

        if with_deg:
            one16 = jnp.ones((16,), jnp.float32)

            def fill_ones(i, carry):
                ones_v[pl.ds(i * 16, 16)] = one16
                return carry
            lax.fori_loop(0, KJ // 16, fill_ones, 0)

        for r in range(nrel):
            soff = pl.multiple_of(s * SLAB, 8)
            pltpu.sync_copy(z2_hbm, rows_0)
            for k in range(SLAB // KJ):
                off = pl.multiple_of(soff + k * KJ, 8)
                pltpu.sync_copy(rows_0, agg_sh.at[pl.ds(off, KJ), :])
            off = pl.multiple_of(soff + (SLAB // KJ) * KJ, 8)
            pltpu.sync_copy(rows_0.at[pl.ds(0, SLAB % KJ), :],
                            agg_sh.at[pl.ds(off, SLAB % KJ), :])

            @pl.when(s == 0)
            def _():
                pltpu.sync_copy(rows_0.at[pl.ds(0, REM), :],
                                agg_sh.at[pl.ds(NS * SLAB, REM), :])
            if with_deg:
                @pl.when(s < 10)
                def _():
                    off = pl.multiple_of(s * 1000, 8)
                    pltpu.sync_copy(z1_hbm.at[pl.ds(0, 1000)],
                                    dstage_v.at[pl.ds(0, 1000)])
                    pltpu.sync_copy(dstage_v.at[pl.ds(0, 1000)],
                                    deg_sh.at[pl.ds(off, 1000)])
            plsc.subcore_barrier()

            def istart(q, p):
                off = pl.multiple_of(r * E + w * EPW + q * EQ, 8)
                blk = (r * NW + w) * NQ + q
                pltpu.make_async_copy(srcf_hbm.at[pl.ds(off, EQ)],
                                      src_q[p], isems[p]).start()
                pltpu.make_async_copy(dstb_hbm.at[blk],
                                      dst_q[p], isems[p]).start()

            def iwait(q, p):
                off = pl.multiple_of(r * E + w * EPW + q * EQ, 8)
                blk = (r * NW + w) * NQ + q
                pltpu.make_async_copy(srcf_hbm.at[pl.ds(off, EQ)],
                                      src_q[p], isems[p]).wait()
                pltpu.make_async_copy(dstb_hbm.at[blk],
                                      dst_q[p], isems[p]).wait()

            istart(0, 0)
            for q in range(NQ):
                qp = q % 2
                srcq, dstq = src_q[qp], dst_q[qp]
                iwait(q, qp)
                if q + 1 < NQ:
                    istart(q + 1, 1 - qp)

                def gidx(j):
                    return srcq.at[pl.ds(pl.multiple_of(j * KJ, 8), KJ)]

                def sidx(j):
                    return dstq.at[j]

                def gstart(j, b):
                    pltpu.make_async_copy(
                        tab_hbm.at[gidx(j)], bufs[b], gsems[b]).start()

                def gwait(j, b):
                    pltpu.make_async_copy(
                        tab_hbm.at[gidx(j)], bufs[b], gsems[b]).wait()

                def sstart(j, b):
                    pltpu.async_copy(bufs[b], agg_sh.at[sidx(j)], ssems[b],
                                     add=True)
                    if with_deg:
                        pltpu.async_copy(ones_v, deg_sh.at[sidx(j)], dsem,
                                         add=True)

                def swait(j, b):
                    pltpu.make_async_copy(
                        bufs[b], agg_sh.at[sidx(j)], ssems[b]).wait()

                gstart(0, 0)
                gstart(1, 1)

                def step(j, b):
                    gwait(j, b)
                    sstart(j, b)
                    nb = (b + 2) % 3

                    @pl.when(j >= 1)
                    def _():
                        swait(j - 1, nb)

                    @pl.when(j + 2 < CQ)
                    def _():
                        gstart(j + 2, nb)

                def body(j, carry):
                    for b in range(3):
                        @pl.when(j % 3 == b)
                        def _():
                            step(j, b)
                    return carry
                lax.fori_loop(0, CQ, body, 0)

                swait(CQ - 1, (CQ - 1) % 3)

                if with_deg:
                    def ddrain(j, carry):
                        pltpu.make_async_copy(ones_v, deg_sh.at[sidx(j)],
                                              dsem).wait()
                        return carry
                    lax.fori_loop(0, CQ, ddrain, 0)

            plsc.subcore_barrier()

            pltpu.sync_copy(agg_sh.at[pl.ds(soff, SLAB), :],
                            agg_out.at[r, c, pl.ds(soff, SLAB), :])

            @pl.when(s == 0)
            def _():
                pltpu.sync_copy(agg_sh.at[pl.ds(NS * SLAB, REM), :],
                                agg_out.at[r, c, pl.ds(NS * SLAB, REM), :])
            if with_deg:
                @pl.when(s < 10)
                def _():
                    off = pl.multiple_of(s * 1000, 8)
                    doff = pl.multiple_of((r * NC + c) * N + s * 1000, 8)
                    pltpu.sync_copy(deg_sh.at[pl.ds(off, 1000)],
                                    dstage_v.at[pl.ds(0, 1000)])
                    pltpu.sync_copy(dstage_v.at[pl.ds(0, 1000)],
                                    deg_out.at[pl.ds(doff, 1000)])
            if r + 1 < nrel:
                plsc.subcore_barrier()

    return sc_agg


_sc_agg3 = _build_sc_agg(3, True)
_sc_agg1 = _build_sc_agg(1, False)

RB = 1000


def _tc_layer1(ap, dp, x, wl, bl, wr):
    def body(ap_ref, dp_ref, x_ref, wl_ref, bl_ref, wr_ref, o_ref):
        xb = x_ref[...]
        acc = jnp.zeros((RB, D), jnp.float32)
        for r in range(3):
            agg = ap_ref[r, 0] + ap_ref[r, 1]
            deg = jnp.maximum(dp_ref[r, 0] + dp_ref[r, 1], 1.0)
            agg = agg / deg
            v = (jnp.dot(agg, wl_ref[r], preferred_element_type=jnp.float32)
                 + jnp.dot(xb, wr_ref[r], preferred_element_type=jnp.float32)
                 + bl_ref[r][None, :])
            acc = acc + jnp.maximum(v, 0.0)
        o_ref[...] = acc * (1.0 / 3.0)

    return pl.pallas_call(
        body,
        grid=(N // RB,),
        in_specs=[
            pl.BlockSpec((3, NC, RB, D), lambda i: (0, 0, i, 0)),
            pl.BlockSpec((3, NC, RB, 1), lambda i: (0, 0, i, 0)),
            pl.BlockSpec((RB, D), lambda i: (i, 0)),
            pl.BlockSpec((3, D, D), lambda i: (0, 0, 0)),
            pl.BlockSpec((3, D), lambda i: (0, 0)),
            pl.BlockSpec((3, D, D), lambda i: (0, 0, 0)),
        ],
        out_specs=pl.BlockSpec((RB, D), lambda i: (i, 0)),
        out_shape=jax.ShapeDtypeStruct((N, D), jnp.float32),
    )(ap, dp, x, wl, bl, wr)


def _tc_layer2(af, dg, h, wlf, blf, wrf):
    def body(af_ref, dg_ref, h_ref, wl_ref, bl_ref, wr_ref, o_ref):
        agg = af_ref[0] + af_ref[1]
        deg = jnp.maximum(dg_ref[0] + dg_ref[1], 1.0)
        agg = agg / deg
        o_ref[...] = (jnp.dot(agg, wl_ref[...], preferred_element_type=jnp.float32)
                      + jnp.dot(h_ref[...], wr_ref[...], preferred_element_type=jnp.float32)
                      + bl_ref[...])

    return pl.pallas_call(
        body,
        grid=(N // RB,),
        in_specs=[
            pl.BlockSpec((NC, RB, D), lambda i: (0, i, 0)),
            pl.BlockSpec((NC, RB, 1), lambda i: (0, i, 0)),
            pl.BlockSpec((RB, D), lambda i: (i, 0)),
            pl.BlockSpec((D, D), lambda i: (0, 0)),
            pl.BlockSpec((1, D), lambda i: (0, 0)),
            pl.BlockSpec((D, D), lambda i: (0, 0)),
        ],
        out_specs=pl.BlockSpec((RB, D), lambda i: (i, 0)),
        out_shape=jax.ShapeDtypeStruct((N, D), jnp.float32),
    )(af, dg, h, wlf, blf, wrf)


def kernel(x, edge_index_0, edge_index_1, edge_index_2,
           W_l_0, b_l_0, W_r_0,
           W_l_1, b_l_1, W_r_1,
           W_l_2, b_l_2, W_r_2,
           W_l_f, b_l_f, W_r_f):
    z2 = jnp.zeros((KJ, D), jnp.float32)
    z1 = jnp.zeros((1024,), jnp.float32)

    e_all = jnp.stack([edge_index_0, edge_index_1, edge_index_2])
    e_all = e_all.astype(jnp.int32)
    src_flat = e_all[:, 0, :].reshape(3 * E)
    dst_blk = e_all[:, 1, :].reshape(3 * NW * NQ, CQ, KJ)
    agg3, deg3 = _sc_agg3(x, z2, z1, src_flat, dst_blk)
    deg3 = deg3.reshape(3, NC, N, 1)

    wl = jnp.stack([W_l_0, W_l_1, W_l_2])
    bl = jnp.stack([b_l_0, b_l_1, b_l_2])
    wr = jnp.stack([W_r_0, W_r_1, W_r_2])
    h = _tc_layer1(agg3, deg3, x, wl, bl, wr)

    ef = edge_index_0.astype(jnp.int32)
    (aggf,) = _sc_agg1(h, z2, z1, ef[0],
                       ef[1].reshape(NW * NQ, CQ, KJ))

    out = _tc_layer2(aggf[0], deg3[0], h,
                     W_l_f, b_l_f.reshape(1, D), W_r_f)
    return out
